# Initial kernel scaffold; baseline (speedup 1.0000x reference)
import jax
import jax.numpy as jnp
from jax.experimental import pallas as pl


def kernel(x_T, weight_VxD):
    raise NotImplementedError("write your pallas kernel here")


# speedup vs baseline: 1.5614x; 1.5614x over previous
import functools

import jax
import jax.numpy as jnp
from jax import lax
from jax.experimental import pallas as pl
from jax.experimental.pallas import tpu as pltpu, tpu_sc as plsc


@functools.cache
def _build_gather(B, V, D):
    info = plsc.get_sparse_core_info()
    NC, NS = info.num_cores, info.num_subcores
    NW = NC * NS
    assert B % NW == 0
    b_per_w = B // NW
    n_chunks = 8
    C = b_per_w // n_chunks
    assert C * n_chunks == b_per_w and C % 8 == 0

    mesh = plsc.VectorSubcoreMesh(core_axis_name="c", subcore_axis_name="s")

    @functools.partial(
        pl.kernel,
        mesh=mesh,
        compiler_params=pltpu.CompilerParams(use_tc_tiling_on_sc=False),
        out_type=jax.ShapeDtypeStruct((B, D), jnp.float32),
        scratch_types=[
            pltpu.VMEM((C,), jnp.int32),
            pltpu.VMEM((C, D), jnp.float32),
            pltpu.SemaphoreType.DMA,
        ],
    )
    def gather_kernel(table_hbm, idx_hbm, out_hbm, idx_v, rows_v, sem):
        wid = lax.axis_index("s") * NC + lax.axis_index("c")
        w_base = wid * b_per_w

        def body(i, carry):
            base = w_base + i * C
            pltpu.sync_copy(idx_hbm.at[pl.ds(base, C)], idx_v)
            pltpu.async_copy(table_hbm.at[idx_v], rows_v, sem).wait()
            pltpu.sync_copy(rows_v, out_hbm.at[pl.ds(base, C)])
            return carry

        lax.fori_loop(0, n_chunks, body, 0)

    return gather_kernel


def kernel(x_T, weight_VxD):
    T, S = x_T.shape
    V, D = weight_VxD.shape
    B = T * S
    flat_idx = x_T.reshape(B)
    out = _build_gather(B, V, D)(weight_VxD, flat_idx)
    return out.reshape(T, S, D)


# speedup vs baseline: 1.5618x; 1.0002x over previous
import functools

import jax
import jax.numpy as jnp
from jax import lax
from jax.experimental import pallas as pl
from jax.experimental.pallas import tpu as pltpu, tpu_sc as plsc


@functools.cache
def _build_gather(B, V, D):
    info = plsc.get_sparse_core_info()
    NC, NS = info.num_cores, info.num_subcores
    NW = NC * NS
    assert B % NW == 0
    b_per_w = B // NW
    n_chunks = 16
    C = b_per_w // n_chunks
    assert C * n_chunks == b_per_w and C % 8 == 0

    mesh = plsc.VectorSubcoreMesh(core_axis_name="c", subcore_axis_name="s")

    @functools.partial(
        pl.kernel,
        mesh=mesh,
        compiler_params=pltpu.CompilerParams(use_tc_tiling_on_sc=False),
        out_type=jax.ShapeDtypeStruct((B, D), jnp.float32),
        scratch_types=[
            pltpu.VMEM((n_chunks, C), jnp.int32),
            pltpu.VMEM((2, C, D), jnp.float32),
            pltpu.SemaphoreType.DMA,
            pltpu.SemaphoreType.DMA,
            pltpu.SemaphoreType.DMA,
            pltpu.SemaphoreType.DMA,
        ],
    )
    def gather_kernel(table_hbm, idx_hbm, out_hbm, idx_v, rows_v, g0, g1, w0, w1):
        wid = lax.axis_index("s") * NC + lax.axis_index("c")
        w_base = wid * b_per_w
        gsem = (g0, g1)
        wsem = (w0, w1)

        pltpu.sync_copy(idx_hbm.at[pl.ds(wid * n_chunks, n_chunks)], idx_v)

        def start_gather(i, b):
            pltpu.async_copy(table_hbm.at[idx_v.at[i]], rows_v.at[b], gsem[b])

        def start_write(i, b):
            pltpu.async_copy(
                rows_v.at[b], out_hbm.at[pl.ds(w_base + i * C, C)], wsem[b]
            )

        start_gather(0, 0)
        for i in range(n_chunks):
            b = i % 2
            pltpu.make_async_copy(
                table_hbm.at[idx_v.at[i]], rows_v.at[b], gsem[b]
            ).wait()
            if i + 1 < n_chunks:
                if i >= 1:
                    pltpu.make_async_copy(
                        rows_v.at[1 - b],
                        out_hbm.at[pl.ds(w_base + (i - 1) * C, C)],
                        wsem[1 - b],
                    ).wait()
                start_gather(i + 1, 1 - b)
            start_write(i, b)
        for i in (n_chunks - 2, n_chunks - 1):
            b = i % 2
            pltpu.make_async_copy(
                rows_v.at[b], out_hbm.at[pl.ds(w_base + i * C, C)], wsem[b]
            ).wait()

    return gather_kernel


def kernel(x_T, weight_VxD):
    T, S = x_T.shape
    V, D = weight_VxD.shape
    B = T * S
    info = plsc.get_sparse_core_info()
    NW = info.num_cores * info.num_subcores
    n_chunks = 16
    C = B // (NW * n_chunks)
    flat_idx = x_T.reshape(NW * n_chunks, C)
    out = _build_gather(B, V, D)(weight_VxD, flat_idx)
    return out.reshape(T, S, D)
